# Initial kernel scaffold; baseline (speedup 1.0000x reference)
import jax
import jax.numpy as jnp
from jax.experimental import pallas as pl


def kernel(inputs, b1_w0, b1_b0, b1_wh, b1_bh, b1_wl, b1_bl, b2_w0, b2_b0, b2_wh, b2_bh, b2_wl, b2_bl, b3_w0, b3_b0, b3_wh, b3_bh, b3_wl, b3_bl, p_array1, p_array2):
    raise NotImplementedError("write your pallas kernel here")


# speedup vs baseline: 1.4029x; 1.4029x over previous
import functools

import jax
import jax.numpy as jnp
from jax import lax
from jax.experimental import pallas as pl
from jax.experimental.pallas import tpu as pltpu


def _elu(x):
    return jnp.where(x > 0, x, jnp.exp(x) - 1.0)


def _enc_kernel(x_ref, w0_ref, b0_ref, wh_ref, bh_ref, wl_ref, bl_ref,
                y_ref, stats_ref, hp_ref, *, n_hidden, ks):
    TB, L, CK = x_ref.shape
    Cp = wl_ref.shape[0]
    pad = ks // 2
    M = TB * L

    xb = (2.0 * x_ref[...] - 1.0).astype(jnp.bfloat16)
    zz = jnp.zeros((TB, pad, CK), jnp.bfloat16)
    xp = jnp.concatenate([zz, xb, zz], axis=1)
    x15 = jnp.concatenate([xp[:, t:t + L, :] for t in range(ks)],
                          axis=-1).reshape(M, ks * CK)
    acc = jnp.dot(x15, w0_ref[...],
                  preferred_element_type=jnp.float32) + b0_ref[...]
    h = _elu(acc)

    hp_ref[:, :pad, :Cp] = jnp.zeros((TB, pad, Cp), jnp.bfloat16)
    hp_ref[:, pad + L:, :Cp] = jnp.zeros((TB, pad + 1, Cp), jnp.bfloat16)
    hp_ref[:, :pad - 1, Cp:] = jnp.zeros((TB, pad - 1, Cp), jnp.bfloat16)
    hp_ref[:, pad - 1 + L:, Cp:] = jnp.zeros((TB, pad + 2, Cp), jnp.bfloat16)

    for layer in range(n_hidden):
        hb = h.astype(jnp.bfloat16).reshape(TB, L, Cp)
        hp_ref[:, pad:pad + L, :Cp] = hb
        hp_ref[:, pad - 1:pad - 1 + L, Cp:] = hb
        l01 = hp_ref[:, 0:L, :].reshape(M, 2 * Cp)
        l23 = hp_ref[:, 2:2 + L, :].reshape(M, 2 * Cp)
        l4 = hp_ref[:, 4:4 + L, :Cp].reshape(M, Cp)
        acc = (jnp.dot(l01, wh_ref[layer, 0:2 * Cp],
                       preferred_element_type=jnp.float32)
               + jnp.dot(l23, wh_ref[layer, 2 * Cp:4 * Cp],
                         preferred_element_type=jnp.float32)
               + jnp.dot(l4, wh_ref[layer, 4 * Cp:5 * Cp],
                         preferred_element_type=jnp.float32)
               + bh_ref[layer])
        h = _elu(acc)

    y = _elu(jnp.dot(h.astype(jnp.bfloat16), wl_ref[...],
                     preferred_element_type=jnp.float32) + bl_ref[...])
    y_ref[...] = y.reshape(TB, L, 3)

    zeros_t = jnp.zeros((8, 128), jnp.float32)
    stats_ref[0, 0] = zeros_t + jnp.sum(y)
    stats_ref[0, 1] = zeros_t + jnp.sum(y * y)


def _norm_kernel(scal_ref, y_ref, out_ref):
    out_ref[...] = (y_ref[...] - scal_ref[0]) * scal_ref[1]


def _pack_params(branches, c_pad):
    ks, K, C = branches[0][0].shape
    n_hidden = branches[0][2].shape[0]
    w0 = jnp.zeros((ks, 3 * K, c_pad), jnp.float32)
    b0 = jnp.zeros((1, c_pad), jnp.float32)
    wh = jnp.zeros((n_hidden, ks, c_pad, c_pad), jnp.float32)
    bh = jnp.zeros((n_hidden, 1, c_pad), jnp.float32)
    wl = jnp.zeros((c_pad, 3), jnp.float32)
    bl = jnp.zeros((1, 3), jnp.float32)
    for r, (w0_r, b0_r, wh_r, bh_r, wl_r, bl_r) in enumerate(branches):
        w0 = w0.at[:, r * K:(r + 1) * K, r * C:(r + 1) * C].set(w0_r)
        b0 = b0.at[:, r * C:(r + 1) * C].set(b0_r)
        wh = wh.at[:, :, r * C:(r + 1) * C, r * C:(r + 1) * C].set(wh_r)
        bh = bh.at[:, 0, r * C:(r + 1) * C].set(bh_r)
        wl = wl.at[r * C:(r + 1) * C, r:r + 1].set(wl_r)
        bl = bl.at[:, r:r + 1].set(bl_r)
    w0cat = w0.reshape(ks * 3 * K, c_pad).astype(jnp.bfloat16)
    whcat = jnp.concatenate([wh[:, t] for t in range(ks)],
                            axis=1).astype(jnp.bfloat16)
    return (w0cat, b0, whcat, bh, wl.astype(jnp.bfloat16), bl,
            n_hidden, ks)


def kernel(inputs,
           b1_w0, b1_b0, b1_wh, b1_bh, b1_wl, b1_bl,
           b2_w0, b2_b0, b2_wh, b2_bh, b2_wl, b2_bl,
           b3_w0, b3_b0, b3_wh, b3_bh, b3_wl, b3_bl,
           p_array1, p_array2):
    B, L, K = inputs.shape
    c_pad = 128
    branches = ((b1_w0, b1_b0, b1_wh, b1_bh, b1_wl, b1_bl),
                (b2_w0, b2_b0, b2_wh, b2_bh, b2_wl, b2_bl),
                (b3_w0, b3_b0, b3_wh, b3_bh, b3_wl, b3_bl))
    w0cat, b0, whcat, bh, wl, bl, n_hidden, ks = _pack_params(branches, c_pad)
    pad = ks // 2

    tile_b = 64
    while B % tile_b:
        tile_b -= 1
    num_tiles = B // tile_b

    x = inputs.astype(jnp.float32)
    x_packed = jnp.concatenate(
        [x, jnp.take(x, p_array1, axis=1), jnp.take(x, p_array2, axis=1)],
        axis=2)

    m_rows = tile_b * L
    flops = 2 * B * L * (ks * 3 * K * c_pad + n_hidden * ks * c_pad * c_pad
                         + c_pad * 3)
    transcendentals = B * L * (c_pad * (1 + n_hidden) + 3)
    bytes_accessed = 4 * (x_packed.size + 2 * B * L * 3
                          + num_tiles * 2 * 8 * 128) + 2 * (
                              w0cat.size + whcat.size + wl.size)

    _fn = functools.partial(_enc_kernel, n_hidden=n_hidden, ks=ks)
    y, stats = pl.pallas_call(
        _fn,
        grid=(num_tiles,),
        in_specs=[
            pl.BlockSpec((tile_b, L, 3 * K), lambda i: (i, 0, 0)),
            pl.BlockSpec(w0cat.shape, lambda i: (0, 0)),
            pl.BlockSpec(b0.shape, lambda i: (0, 0)),
            pl.BlockSpec(whcat.shape, lambda i: (0, 0, 0)),
            pl.BlockSpec(bh.shape, lambda i: (0, 0, 0)),
            pl.BlockSpec(wl.shape, lambda i: (0, 0)),
            pl.BlockSpec(bl.shape, lambda i: (0, 0)),
        ],
        out_shape=(
            jax.ShapeDtypeStruct((B, L, 3), jnp.float32),
            jax.ShapeDtypeStruct((num_tiles, 2, 8, 128), jnp.float32),
        ),
        out_specs=(
            pl.BlockSpec((tile_b, L, 3), lambda i: (i, 0, 0)),
            pl.BlockSpec((1, 2, 8, 128), lambda i: (i, 0, 0, 0)),
        ),
        scratch_shapes=[
            pltpu.VMEM((tile_b, L + 2 * pad + 1, 2 * c_pad), jnp.bfloat16)],
        compiler_params=pltpu.CompilerParams(
            dimension_semantics=("parallel",),
            vmem_limit_bytes=48 * 2 ** 20),
        cost_estimate=pl.CostEstimate(flops=int(flops),
                                      transcendentals=int(transcendentals),
                                      bytes_accessed=int(bytes_accessed)),
    )(x_packed, w0cat, b0, whcat, bh, wl, bl)

    n = float(B * L * 3)
    total = jnp.sum(stats[:, 0, 0, 0])
    total_sq = jnp.sum(stats[:, 1, 0, 0])
    mean = total / n
    var = (total_sq - n * mean * mean) / (n - 1.0)
    scal = jnp.stack([mean, lax.rsqrt(var)]).astype(jnp.float32)

    y2 = y.reshape(B, L * 3)
    tile_n = 1024
    while B % tile_n:
        tile_n -= 1
    codes = pl.pallas_call(
        _norm_kernel,
        grid=(B // tile_n,),
        in_specs=[
            pl.BlockSpec((2,), lambda i: (0,),
                         memory_space=pltpu.MemorySpace.SMEM),
            pl.BlockSpec((tile_n, L * 3), lambda i: (i, 0)),
        ],
        out_shape=jax.ShapeDtypeStruct((B, L * 3), jnp.float32),
        out_specs=pl.BlockSpec((tile_n, L * 3), lambda i: (i, 0)),
        compiler_params=pltpu.CompilerParams(
            dimension_semantics=("parallel",)),
    )(scal, y2)
    return codes.reshape(B, L, 3)


# speedup vs baseline: 3.0832x; 2.1977x over previous
import functools

import jax
import jax.numpy as jnp
from jax import lax
from jax.experimental import pallas as pl
from jax.experimental.pallas import tpu as pltpu


def _elu(x):
    return jnp.where(x > 0, x, jnp.exp(x) - 1.0)


def _enc_kernel(x_ref, perm_ref, w0_ref, b0_ref, wh_ref, bh_ref, wl_ref,
                bl_ref, y_ref, stats_ref, hp_ref, *, n_hidden, ks):
    TB, L = x_ref.shape
    CK = 3
    Cp = wl_ref.shape[0]
    pad = ks // 2
    M = TB * L

    a = (2.0 * x_ref[...] - 1.0).astype(jnp.bfloat16)
    ai1 = jnp.dot(a, perm_ref[0],
                  preferred_element_type=jnp.float32).astype(jnp.bfloat16)
    ai2 = jnp.dot(a, perm_ref[1],
                  preferred_element_type=jnp.float32).astype(jnp.bfloat16)
    xb = jnp.stack([a, ai1, ai2], axis=-1)

    zz = jnp.zeros((TB, pad, CK), jnp.bfloat16)
    xp = jnp.concatenate([zz, xb, zz], axis=1)
    x15 = jnp.concatenate([xp[:, t:t + L, :] for t in range(ks)],
                          axis=-1).reshape(M, ks * CK)
    acc = jnp.dot(x15, w0_ref[...],
                  preferred_element_type=jnp.float32) + b0_ref[...]
    h = _elu(acc)

    hp_ref[:, :pad, :Cp] = jnp.zeros((TB, pad, Cp), jnp.bfloat16)
    hp_ref[:, pad + L:, :Cp] = jnp.zeros((TB, pad + 1, Cp), jnp.bfloat16)
    hp_ref[:, :pad - 1, Cp:] = jnp.zeros((TB, pad - 1, Cp), jnp.bfloat16)
    hp_ref[:, pad - 1 + L:, Cp:] = jnp.zeros((TB, pad + 2, Cp), jnp.bfloat16)

    for layer in range(n_hidden):
        hb = h.astype(jnp.bfloat16).reshape(TB, L, Cp)
        hp_ref[:, pad:pad + L, :Cp] = hb
        hp_ref[:, pad - 1:pad - 1 + L, Cp:] = hb
        l01 = hp_ref[:, 0:L, :].reshape(M, 2 * Cp)
        l23 = hp_ref[:, 2:2 + L, :].reshape(M, 2 * Cp)
        l4 = hp_ref[:, 4:4 + L, :Cp].reshape(M, Cp)
        acc = (jnp.dot(l01, wh_ref[layer, 0:2 * Cp],
                       preferred_element_type=jnp.float32)
               + jnp.dot(l23, wh_ref[layer, 2 * Cp:4 * Cp],
                         preferred_element_type=jnp.float32)
               + jnp.dot(l4, wh_ref[layer, 4 * Cp:5 * Cp],
                         preferred_element_type=jnp.float32)
               + bh_ref[layer])
        h = _elu(acc)

    y = _elu(jnp.dot(h.astype(jnp.bfloat16), wl_ref[...],
                     preferred_element_type=jnp.float32) + bl_ref[...])
    y_ref[...] = y.reshape(TB, L, 3)

    zeros_t = jnp.zeros((8, 128), jnp.float32)
    stats_ref[0, 0] = zeros_t + jnp.sum(y)
    stats_ref[0, 1] = zeros_t + jnp.sum(y * y)


def _norm_kernel(scal_ref, y_ref, out_ref):
    out_ref[...] = (y_ref[...] - scal_ref[0]) * scal_ref[1]


def _pack_params(branches, c_pad):
    ks, K, C = branches[0][0].shape
    n_hidden = branches[0][2].shape[0]
    w0 = jnp.zeros((ks, 3 * K, c_pad), jnp.float32)
    b0 = jnp.zeros((1, c_pad), jnp.float32)
    wh = jnp.zeros((n_hidden, ks, c_pad, c_pad), jnp.float32)
    bh = jnp.zeros((n_hidden, 1, c_pad), jnp.float32)
    wl = jnp.zeros((c_pad, 3), jnp.float32)
    bl = jnp.zeros((1, 3), jnp.float32)
    for r, (w0_r, b0_r, wh_r, bh_r, wl_r, bl_r) in enumerate(branches):
        w0 = w0.at[:, r * K:(r + 1) * K, r * C:(r + 1) * C].set(w0_r)
        b0 = b0.at[:, r * C:(r + 1) * C].set(b0_r)
        wh = wh.at[:, :, r * C:(r + 1) * C, r * C:(r + 1) * C].set(wh_r)
        bh = bh.at[:, 0, r * C:(r + 1) * C].set(bh_r)
        wl = wl.at[r * C:(r + 1) * C, r:r + 1].set(wl_r)
        bl = bl.at[:, r:r + 1].set(bl_r)
    w0cat = w0.reshape(ks * 3 * K, c_pad).astype(jnp.bfloat16)
    whcat = jnp.concatenate([wh[:, t] for t in range(ks)],
                            axis=1).astype(jnp.bfloat16)
    return (w0cat, b0, whcat, bh, wl.astype(jnp.bfloat16), bl,
            n_hidden, ks)


def kernel(inputs,
           b1_w0, b1_b0, b1_wh, b1_bh, b1_wl, b1_bl,
           b2_w0, b2_b0, b2_wh, b2_bh, b2_wl, b2_bl,
           b3_w0, b3_b0, b3_wh, b3_bh, b3_wl, b3_bl,
           p_array1, p_array2):
    B, L, K = inputs.shape
    c_pad = 128
    branches = ((b1_w0, b1_b0, b1_wh, b1_bh, b1_wl, b1_bl),
                (b2_w0, b2_b0, b2_wh, b2_bh, b2_wl, b2_bl),
                (b3_w0, b3_b0, b3_wh, b3_bh, b3_wl, b3_bl))
    w0cat, b0, whcat, bh, wl, bl, n_hidden, ks = _pack_params(branches, c_pad)
    pad = ks // 2

    tile_b = 64
    while B % tile_b:
        tile_b -= 1
    num_tiles = B // tile_b

    x2 = inputs.astype(jnp.float32).reshape(B, L)
    lidx = jnp.arange(L, dtype=jnp.int32)
    perm = jnp.stack([
        (lidx[:, None] == p_array1[None, :]).astype(jnp.bfloat16),
        (lidx[:, None] == p_array2[None, :]).astype(jnp.bfloat16)])

    flops = 2 * B * L * (ks * 3 * K * c_pad + n_hidden * ks * c_pad * c_pad
                         + c_pad * 3 + 2 * L)
    transcendentals = B * L * (c_pad * (1 + n_hidden) + 3)
    bytes_accessed = 4 * (x2.size + 2 * B * L * 3
                          + num_tiles * 2 * 8 * 128) + 2 * (
                              w0cat.size + whcat.size + wl.size)

    _fn = functools.partial(_enc_kernel, n_hidden=n_hidden, ks=ks)
    y, stats = pl.pallas_call(
        _fn,
        grid=(num_tiles,),
        in_specs=[
            pl.BlockSpec((tile_b, L), lambda i: (i, 0)),
            pl.BlockSpec((2, L, L), lambda i: (0, 0, 0)),
            pl.BlockSpec(w0cat.shape, lambda i: (0, 0)),
            pl.BlockSpec(b0.shape, lambda i: (0, 0)),
            pl.BlockSpec(whcat.shape, lambda i: (0, 0, 0)),
            pl.BlockSpec(bh.shape, lambda i: (0, 0, 0)),
            pl.BlockSpec(wl.shape, lambda i: (0, 0)),
            pl.BlockSpec(bl.shape, lambda i: (0, 0)),
        ],
        out_shape=(
            jax.ShapeDtypeStruct((B, L, 3), jnp.float32),
            jax.ShapeDtypeStruct((num_tiles, 2, 8, 128), jnp.float32),
        ),
        out_specs=(
            pl.BlockSpec((tile_b, L, 3), lambda i: (i, 0, 0)),
            pl.BlockSpec((1, 2, 8, 128), lambda i: (i, 0, 0, 0)),
        ),
        scratch_shapes=[
            pltpu.VMEM((tile_b, L + 2 * pad + 1, 2 * c_pad), jnp.bfloat16)],
        compiler_params=pltpu.CompilerParams(
            dimension_semantics=("parallel",),
            vmem_limit_bytes=48 * 2 ** 20),
        cost_estimate=pl.CostEstimate(flops=int(flops),
                                      transcendentals=int(transcendentals),
                                      bytes_accessed=int(bytes_accessed)),
    )(x2, perm, w0cat, b0, whcat, bh, wl, bl)

    n = float(B * L * 3)
    total = jnp.sum(stats[:, 0, 0, 0])
    total_sq = jnp.sum(stats[:, 1, 0, 0])
    mean = total / n
    var = (total_sq - n * mean * mean) / (n - 1.0)
    scal = jnp.stack([mean, lax.rsqrt(var)]).astype(jnp.float32)

    y2 = y.reshape(B, L * 3)
    tile_n = 1024
    while B % tile_n:
        tile_n -= 1
    codes = pl.pallas_call(
        _norm_kernel,
        grid=(B // tile_n,),
        in_specs=[
            pl.BlockSpec((2,), lambda i: (0,),
                         memory_space=pltpu.MemorySpace.SMEM),
            pl.BlockSpec((tile_n, L * 3), lambda i: (i, 0)),
        ],
        out_shape=jax.ShapeDtypeStruct((B, L * 3), jnp.float32),
        out_specs=pl.BlockSpec((tile_n, L * 3), lambda i: (i, 0)),
        compiler_params=pltpu.CompilerParams(
            dimension_semantics=("parallel",)),
    )(scal, y2)
    return codes.reshape(B, L, 3)


# speedup vs baseline: 3.5472x; 1.1505x over previous
import functools

import jax
import jax.numpy as jnp
from jax import lax
from jax.experimental import pallas as pl
from jax.experimental.pallas import tpu as pltpu


def _elu(x):
    return jnp.where(x > 0, x, jnp.exp(x) - 1.0)


def _enc_kernel(x_ref, pp_ref, w0_ref, b0_ref, wh_ref, bh_ref, wl_ref,
                bl_ref, y_ref, stats_ref, hq_ref, *, n_hidden):
    TB, L = x_ref.shape
    L2 = L // 2
    Cp2 = w0_ref.shape[1]
    M2 = TB * L2

    a = (2.0 * x_ref[...] - 1.0).astype(jnp.bfloat16)
    planes = jnp.dot(a, pp_ref[...],
                     preferred_element_type=jnp.float32)
    xs6 = jnp.stack([planes[:, p * L2:(p + 1) * L2] for p in range(6)],
                    axis=-1).astype(jnp.bfloat16)
    zz = jnp.zeros((TB, 1, 6), jnp.bfloat16)
    xp6 = jnp.concatenate([zz, xs6, zz], axis=1)
    x18 = jnp.concatenate([xp6[:, d:d + L2, :] for d in range(3)],
                          axis=-1).reshape(M2, 18)
    acc = jnp.dot(x18, w0_ref[...],
                  preferred_element_type=jnp.float32) + b0_ref[...]
    eo = _elu(acc).astype(jnp.bfloat16)

    hq_ref[:, 0, :] = jnp.zeros((TB, Cp2), jnp.bfloat16)
    hq_ref[:, L2 + 1, :] = jnp.zeros((TB, Cp2), jnp.bfloat16)
    for layer in range(n_hidden):
        hq_ref[:, 1:L2 + 1, :] = eo.reshape(TB, L2, Cp2)
        lhs = jnp.concatenate([hq_ref[:, d:d + L2, :] for d in range(3)],
                              axis=-1).reshape(M2, 3 * Cp2)
        acc = jnp.dot(lhs, wh_ref[layer],
                      preferred_element_type=jnp.float32) + bh_ref[layer]
        eo = _elu(acc).astype(jnp.bfloat16)

    y = _elu(jnp.dot(eo, wl_ref[...],
                     preferred_element_type=jnp.float32) + bl_ref[...])
    y_ref[...] = y

    zeros_t = jnp.zeros((8, 128), jnp.float32)
    stats_ref[0, 0] = zeros_t + jnp.sum(y)
    stats_ref[0, 1] = zeros_t + jnp.sum(y * y)


def _norm_kernel(scal_ref, y_ref, out_ref):
    out_ref[...] = (y_ref[...] - scal_ref[0]) * scal_ref[1]


def _pack_params(branches, c_pad):
    ks, K, C = branches[0][0].shape
    n_hidden = branches[0][2].shape[0]
    w0 = jnp.zeros((ks, 3, c_pad), jnp.float32)
    b0 = jnp.zeros((1, c_pad), jnp.float32)
    wh = jnp.zeros((n_hidden, ks, c_pad, c_pad), jnp.float32)
    bh = jnp.zeros((n_hidden, 1, c_pad), jnp.float32)
    wl = jnp.zeros((c_pad, 3), jnp.float32)
    bl = jnp.zeros((1, 3), jnp.float32)
    for r, (w0_r, b0_r, wh_r, bh_r, wl_r, bl_r) in enumerate(branches):
        w0 = w0.at[:, r, r * C:(r + 1) * C].set(w0_r[:, 0, :])
        b0 = b0.at[:, r * C:(r + 1) * C].set(b0_r)
        wh = wh.at[:, :, r * C:(r + 1) * C, r * C:(r + 1) * C].set(wh_r)
        bh = bh.at[:, 0, r * C:(r + 1) * C].set(bh_r)
        wl = wl.at[r * C:(r + 1) * C, r:r + 1].set(wl_r)
        bl = bl.at[:, r:r + 1].set(bl_r)

    w0p = jnp.zeros((3, 2, 3, 2 * c_pad), jnp.float32)
    for d in range(3):
        w0p = w0p.at[d, 0, :, :c_pad].set(w0[2 * d])
        w0p = w0p.at[d, 1, :, c_pad:].set(w0[2 * d])
        if d < 2:
            w0p = w0p.at[d, 1, :, :c_pad].set(w0[2 * d + 1])
        if d > 0:
            w0p = w0p.at[d, 0, :, c_pad:].set(w0[2 * d - 1])
    w0p = w0p.reshape(18, 2 * c_pad).astype(jnp.bfloat16)

    whp = jnp.zeros((n_hidden, 3, 2, c_pad, 2 * c_pad), jnp.float32)
    for d in range(3):
        whp = whp.at[:, d, 0, :, :c_pad].set(wh[:, 2 * d])
        whp = whp.at[:, d, 1, :, c_pad:].set(wh[:, 2 * d])
        if d < 2:
            whp = whp.at[:, d, 1, :, :c_pad].set(wh[:, 2 * d + 1])
        if d > 0:
            whp = whp.at[:, d, 0, :, c_pad:].set(wh[:, 2 * d - 1])
    whp = whp.reshape(n_hidden, 6 * c_pad, 2 * c_pad).astype(jnp.bfloat16)

    b0p = jnp.concatenate([b0, b0], axis=1)
    bhp = jnp.concatenate([bh, bh], axis=2)
    wl2 = jnp.zeros((2 * c_pad, 6), jnp.float32)
    wl2 = wl2.at[:c_pad, 0:3].set(wl)
    wl2 = wl2.at[c_pad:, 3:6].set(wl)
    blp = jnp.concatenate([bl, bl], axis=1)
    return (w0p, b0p, whp, bhp, wl2.astype(jnp.bfloat16), blp, n_hidden)


def kernel(inputs,
           b1_w0, b1_b0, b1_wh, b1_bh, b1_wl, b1_bl,
           b2_w0, b2_b0, b2_wh, b2_bh, b2_wl, b2_bl,
           b3_w0, b3_b0, b3_wh, b3_bh, b3_wl, b3_bl,
           p_array1, p_array2):
    B, L, K = inputs.shape
    L2 = L // 2
    c_pad = 128
    branches = ((b1_w0, b1_b0, b1_wh, b1_bh, b1_wl, b1_bl),
                (b2_w0, b2_b0, b2_wh, b2_bh, b2_wl, b2_bl),
                (b3_w0, b3_b0, b3_wh, b3_bh, b3_wl, b3_bl))
    w0p, b0p, whp, bhp, wl2, blp, n_hidden = _pack_params(branches, c_pad)

    tile_b = 64
    while B % tile_b:
        tile_b -= 1
    num_tiles = B // tile_b
    M2 = tile_b * L2

    x2 = inputs.astype(jnp.float32).reshape(B, L)
    lidx = jnp.arange(L, dtype=jnp.int32)
    ident = lidx
    perms = (ident, p_array1, p_array2)
    cols = []
    for half in range(2):
        for p in range(3):
            cols.append(perms[p][half::2])
    src = jnp.concatenate(cols)
    pp = (lidx[:, None] == src[None, :]).astype(jnp.bfloat16)

    flops = 2 * B * (L * 6 * L2 + L2 * (18 * 2 * c_pad
                     + n_hidden * 6 * c_pad * 2 * c_pad + 2 * c_pad * 6))
    transcendentals = B * L * (c_pad * (1 + n_hidden) + 3)
    bytes_accessed = 4 * (x2.size + 2 * B * L * 3
                          + num_tiles * 2 * 8 * 128) + 2 * (
                              w0p.size + whp.size + wl2.size + pp.size)

    _fn = functools.partial(_enc_kernel, n_hidden=n_hidden)
    y2, stats = pl.pallas_call(
        _fn,
        grid=(num_tiles,),
        in_specs=[
            pl.BlockSpec((tile_b, L), lambda i: (i, 0)),
            pl.BlockSpec(pp.shape, lambda i: (0, 0)),
            pl.BlockSpec(w0p.shape, lambda i: (0, 0)),
            pl.BlockSpec(b0p.shape, lambda i: (0, 0)),
            pl.BlockSpec(whp.shape, lambda i: (0, 0, 0)),
            pl.BlockSpec(bhp.shape, lambda i: (0, 0, 0)),
            pl.BlockSpec(wl2.shape, lambda i: (0, 0)),
            pl.BlockSpec(blp.shape, lambda i: (0, 0)),
        ],
        out_shape=(
            jax.ShapeDtypeStruct((B * L2, 6), jnp.float32),
            jax.ShapeDtypeStruct((num_tiles, 2, 8, 128), jnp.float32),
        ),
        out_specs=(
            pl.BlockSpec((M2, 6), lambda i: (i, 0)),
            pl.BlockSpec((1, 2, 8, 128), lambda i: (i, 0, 0, 0)),
        ),
        scratch_shapes=[
            pltpu.VMEM((tile_b, L2 + 2, 2 * c_pad), jnp.bfloat16)],
        compiler_params=pltpu.CompilerParams(
            dimension_semantics=("parallel",),
            vmem_limit_bytes=48 * 2 ** 20),
        cost_estimate=pl.CostEstimate(flops=int(flops),
                                      transcendentals=int(transcendentals),
                                      bytes_accessed=int(bytes_accessed)),
    )(x2, pp, w0p, b0p, whp, bhp, wl2, blp)

    n = float(B * L * 3)
    total = jnp.sum(stats[:, 0, 0, 0])
    total_sq = jnp.sum(stats[:, 1, 0, 0])
    mean = total / n
    var = (total_sq - n * mean * mean) / (n - 1.0)
    scal = jnp.stack([mean, lax.rsqrt(var)]).astype(jnp.float32)

    nflat = B * L * 3
    wide = 1024
    while nflat % wide:
        wide //= 2
    rows = nflat // wide
    yw = y2.reshape(rows, wide)
    tile_n = 1024
    while rows % tile_n:
        tile_n -= 1
    codes = pl.pallas_call(
        _norm_kernel,
        grid=(rows // tile_n,),
        in_specs=[
            pl.BlockSpec((2,), lambda i: (0,),
                         memory_space=pltpu.MemorySpace.SMEM),
            pl.BlockSpec((tile_n, wide), lambda i: (i, 0)),
        ],
        out_shape=jax.ShapeDtypeStruct((rows, wide), jnp.float32),
        out_specs=pl.BlockSpec((tile_n, wide), lambda i: (i, 0)),
        compiler_params=pltpu.CompilerParams(
            dimension_semantics=("parallel",)),
    )(scal, yw)
    return codes.reshape(B, L, 3)


# speedup vs baseline: 3.9283x; 1.1074x over previous
import functools

import jax
import jax.numpy as jnp
from jax import lax
from jax.experimental import pallas as pl
from jax.experimental.pallas import tpu as pltpu

_P = 8


def _elu(x):
    return jnp.where(x > 0, x, jnp.exp(x) - 1.0)


def _shift_edges(v, lo_lanes, hi_lanes, J):
    R = v.shape[0]
    j_iota = lax.broadcasted_iota(jnp.int32, (R, 1), 0) % J
    hi = v[:, hi_lanes[0]:hi_lanes[1]]
    lo = v[:, lo_lanes[0]:lo_lanes[1]]
    zrow_h = jnp.zeros((1,) + hi.shape[1:], v.dtype)
    zrow_l = jnp.zeros((1,) + lo.shape[1:], v.dtype)
    prev_hi = jnp.concatenate([zrow_h, hi[:-1]], axis=0)
    next_lo = jnp.concatenate([lo[1:], zrow_l], axis=0)
    prev_hi = jnp.where(j_iota == 0, 0.0, prev_hi)
    next_lo = jnp.where(j_iota == J - 1, 0.0, next_lo)
    return prev_hi, next_lo


def _enc_kernel(x_ref, pp_ref, w00_ref, w0m_ref, w0p_ref, b0_ref,
                wh0_ref, whm_ref, whp_ref, bh_ref, wl_ref, bl_ref,
                y_ref, stats_ref, *, n_hidden):
    TB, L = x_ref.shape
    J = L // _P
    R = TB * J
    CpP = wl_ref.shape[0]

    a = (2.0 * x_ref[...] - 1.0).astype(jnp.bfloat16)
    planes = jnp.dot(a, pp_ref[...],
                     preferred_element_type=jnp.float32)
    xs = jnp.stack([planes[:, m * J:(m + 1) * J] for m in range(24)],
                   axis=-1).astype(jnp.bfloat16).reshape(R, 24)
    xm, xp = _shift_edges(xs, (0, 6), (18, 24), J)
    acc = jnp.dot(xs, w00_ref[...], preferred_element_type=jnp.float32)
    accL = jnp.dot(xm, w0m_ref[...], preferred_element_type=jnp.float32)
    accR = jnp.dot(xp, w0p_ref[...], preferred_element_type=jnp.float32)
    acc = jnp.concatenate(
        [acc[:, :256] + accL, acc[:, 256:768], acc[:, 768:] + accR],
        axis=1) + b0_ref[...]
    h = _elu(acc).astype(jnp.bfloat16)

    for layer in range(n_hidden):
        hm, hp = _shift_edges(h, (0, 256), (768, 1024), J)
        acc = jnp.dot(h, wh0_ref[layer], preferred_element_type=jnp.float32)
        accL = jnp.dot(hm, whm_ref[layer], preferred_element_type=jnp.float32)
        accR = jnp.dot(hp, whp_ref[layer], preferred_element_type=jnp.float32)
        acc = jnp.concatenate(
            [acc[:, :256] + accL, acc[:, 256:768], acc[:, 768:] + accR],
            axis=1) + bh_ref[layer]
        h = _elu(acc).astype(jnp.bfloat16)

    y = _elu(jnp.dot(h, wl_ref[...],
                     preferred_element_type=jnp.float32) + bl_ref[...])
    y_ref[...] = y

    zeros_t = jnp.zeros((8, 128), jnp.float32)
    stats_ref[0, 0] = zeros_t + jnp.sum(y)
    stats_ref[0, 1] = zeros_t + jnp.sum(y * y)


def _norm_kernel(scal_ref, y_ref, out_ref):
    out_ref[...] = (y_ref[...] - scal_ref[0]) * scal_ref[1]


def _pack_params(branches, c_pad):
    ks, K, C = branches[0][0].shape
    n_hidden = branches[0][2].shape[0]
    w0 = jnp.zeros((ks, 3, c_pad), jnp.float32)
    b0 = jnp.zeros((1, c_pad), jnp.float32)
    wh = jnp.zeros((n_hidden, ks, c_pad, c_pad), jnp.float32)
    bh = jnp.zeros((n_hidden, 1, c_pad), jnp.float32)
    wl = jnp.zeros((c_pad, 3), jnp.float32)
    bl = jnp.zeros((1, 3), jnp.float32)
    for r, (w0_r, b0_r, wh_r, bh_r, wl_r, bl_r) in enumerate(branches):
        w0 = w0.at[:, r, r * C:(r + 1) * C].set(w0_r[:, 0, :])
        b0 = b0.at[:, r * C:(r + 1) * C].set(b0_r)
        wh = wh.at[:, :, r * C:(r + 1) * C, r * C:(r + 1) * C].set(wh_r)
        bh = bh.at[:, 0, r * C:(r + 1) * C].set(bh_r)
        wl = wl.at[r * C:(r + 1) * C, r:r + 1].set(wl_r)
        bl = bl.at[:, r:r + 1].set(bl_r)

    P = _P
    w00 = jnp.zeros((P, 3, P, c_pad), jnp.float32)
    wh0 = jnp.zeros((n_hidden, P, c_pad, P, c_pad), jnp.float32)
    for q in range(P):
        for p in range(P):
            t = q - p + 2
            if 0 <= t < ks:
                w00 = w00.at[q, :, p, :].set(w0[t])
                wh0 = wh0.at[:, q, :, p, :].set(wh[:, t])
    w0m = jnp.zeros((2, 3, 2, c_pad), jnp.float32)
    whm = jnp.zeros((n_hidden, 2, c_pad, 2, c_pad), jnp.float32)
    w0p = jnp.zeros((2, 3, 2, c_pad), jnp.float32)
    whp = jnp.zeros((n_hidden, 2, c_pad, 2, c_pad), jnp.float32)
    for qq in range(2):
        for p in range(2):
            t = qq - p
            if 0 <= t < 2:
                w0m = w0m.at[qq, :, p, :].set(w0[t])
                whm = whm.at[:, qq, :, p, :].set(wh[:, t])
            t2 = qq + 4 - p
            if 3 <= t2 < ks:
                w0p = w0p.at[qq, :, p, :].set(w0[t2])
                whp = whp.at[:, qq, :, p, :].set(wh[:, t2])

    w00 = w00.reshape(P * 3, P * c_pad).astype(jnp.bfloat16)
    wh0 = wh0.reshape(n_hidden, P * c_pad, P * c_pad).astype(jnp.bfloat16)
    w0m = w0m.reshape(6, 2 * c_pad).astype(jnp.bfloat16)
    whm = whm.reshape(n_hidden, 2 * c_pad, 2 * c_pad).astype(jnp.bfloat16)
    w0p = w0p.reshape(6, 2 * c_pad).astype(jnp.bfloat16)
    whp = whp.reshape(n_hidden, 2 * c_pad, 2 * c_pad).astype(jnp.bfloat16)

    b8 = jnp.tile(b0, (1, P))
    bh8 = jnp.tile(bh, (1, 1, P))
    wl8 = jnp.zeros((P, c_pad, P, 3), jnp.float32)
    for p in range(P):
        wl8 = wl8.at[p, :, p, :].set(wl)
    wl8 = wl8.reshape(P * c_pad, P * 3).astype(jnp.bfloat16)
    bl8 = jnp.tile(bl, (1, P))
    return (w00, w0m, w0p, b8, wh0, whm, whp, bh8, wl8, bl8, n_hidden)


def kernel(inputs,
           b1_w0, b1_b0, b1_wh, b1_bh, b1_wl, b1_bl,
           b2_w0, b2_b0, b2_wh, b2_bh, b2_wl, b2_bl,
           b3_w0, b3_b0, b3_wh, b3_bh, b3_wl, b3_bl,
           p_array1, p_array2):
    B, L, K = inputs.shape
    P = _P
    J = L // P
    c_pad = 128
    branches = ((b1_w0, b1_b0, b1_wh, b1_bh, b1_wl, b1_bl),
                (b2_w0, b2_b0, b2_wh, b2_bh, b2_wl, b2_bl),
                (b3_w0, b3_b0, b3_wh, b3_bh, b3_wl, b3_bl))
    (w00, w0m, w0p, b8, wh0, whm, whp, bh8, wl8, bl8,
     n_hidden) = _pack_params(branches, c_pad)

    tile_b = 64
    while B % tile_b:
        tile_b -= 1
    num_tiles = B // tile_b
    R = tile_b * J

    x2 = inputs.astype(jnp.float32).reshape(B, L)
    lidx = jnp.arange(L, dtype=jnp.int32)
    perms = (lidx, p_array1, p_array2)
    cols = []
    for p in range(P):
        for br in range(3):
            cols.append(perms[br][p::P])
    src = jnp.concatenate(cols)
    pp = (lidx[:, None] == src[None, :]).astype(jnp.bfloat16)

    flops = 2 * B * (L * 24 * J + J * (24 * P * c_pad
                     + n_hidden * (P + 1) * c_pad * P * c_pad
                     + P * c_pad * 24))
    transcendentals = B * L * (c_pad * (1 + n_hidden) + 3)
    bytes_accessed = 4 * (x2.size + 2 * B * L * 3
                          + num_tiles * 2 * 8 * 128) + 2 * (
                              w00.size + wh0.size + wl8.size + pp.size)

    _fn = functools.partial(_enc_kernel, n_hidden=n_hidden)
    y2, stats = pl.pallas_call(
        _fn,
        grid=(num_tiles,),
        in_specs=[
            pl.BlockSpec((tile_b, L), lambda i: (i, 0)),
            pl.BlockSpec(pp.shape, lambda i: (0, 0)),
            pl.BlockSpec(w00.shape, lambda i: (0, 0)),
            pl.BlockSpec(w0m.shape, lambda i: (0, 0)),
            pl.BlockSpec(w0p.shape, lambda i: (0, 0)),
            pl.BlockSpec(b8.shape, lambda i: (0, 0)),
            pl.BlockSpec(wh0.shape, lambda i: (0, 0, 0)),
            pl.BlockSpec(whm.shape, lambda i: (0, 0, 0)),
            pl.BlockSpec(whp.shape, lambda i: (0, 0, 0)),
            pl.BlockSpec(bh8.shape, lambda i: (0, 0, 0)),
            pl.BlockSpec(wl8.shape, lambda i: (0, 0)),
            pl.BlockSpec(bl8.shape, lambda i: (0, 0)),
        ],
        out_shape=(
            jax.ShapeDtypeStruct((B * J, 24), jnp.float32),
            jax.ShapeDtypeStruct((num_tiles, 2, 8, 128), jnp.float32),
        ),
        out_specs=(
            pl.BlockSpec((R, 24), lambda i: (i, 0)),
            pl.BlockSpec((1, 2, 8, 128), lambda i: (i, 0, 0, 0)),
        ),
        compiler_params=pltpu.CompilerParams(
            dimension_semantics=("parallel",),
            vmem_limit_bytes=60 * 2 ** 20),
        cost_estimate=pl.CostEstimate(flops=int(flops),
                                      transcendentals=int(transcendentals),
                                      bytes_accessed=int(bytes_accessed)),
    )(x2, pp, w00, w0m, w0p, b8, wh0, whm, whp, bh8, wl8, bl8)

    n = float(B * L * 3)
    total = jnp.sum(stats[:, 0, 0, 0])
    total_sq = jnp.sum(stats[:, 1, 0, 0])
    mean = total / n
    var = (total_sq - n * mean * mean) / (n - 1.0)
    scal = jnp.stack([mean, lax.rsqrt(var)]).astype(jnp.float32)

    nflat = B * L * 3
    wide = 1024
    while nflat % wide:
        wide //= 2
    rows = nflat // wide
    yw = y2.reshape(rows, wide)
    tile_n = 1024
    while rows % tile_n:
        tile_n -= 1
    codes = pl.pallas_call(
        _norm_kernel,
        grid=(rows // tile_n,),
        in_specs=[
            pl.BlockSpec((2,), lambda i: (0,),
                         memory_space=pltpu.MemorySpace.SMEM),
            pl.BlockSpec((tile_n, wide), lambda i: (i, 0)),
        ],
        out_shape=jax.ShapeDtypeStruct((rows, wide), jnp.float32),
        out_specs=pl.BlockSpec((tile_n, wide), lambda i: (i, 0)),
        compiler_params=pltpu.CompilerParams(
            dimension_semantics=("parallel",)),
    )(scal, yw)
    return codes.reshape(B, L, 3)
